# Initial kernel scaffold; baseline (speedup 1.0000x reference)
import jax
import jax.numpy as jnp
from jax.experimental import pallas as pl


def kernel(h, edge_index, W1, b1, W2, b2, Wr, br):
    raise NotImplementedError("write your pallas kernel here")


# speedup vs baseline: 4.0268x; 4.0268x over previous
import functools

import jax
import jax.numpy as jnp
import numpy as np
from jax import lax
from jax.experimental import pallas as pl
from jax.experimental.pallas import tpu as pltpu
from jax.experimental.pallas import tpu_sc as plsc

N = 10000
E = 320000
D = 128

NC = 2
NS = 16
NW = NC * NS
CHUNK = 128
NCHUNK = 79
EPW = NCHUNK * CHUNK
EPAD = NW * EPW
NPAD = 10240
TROWS = NPAD // NS
DUMMY = NPAD - 1


def _sc_mesh():
    return plsc.VectorSubcoreMesh(core_axis_name="c", subcore_axis_name="s",
                                  num_cores=NC, num_subcores=NS)


def _deg_pass(ei, zrow, onesrc, onedst):
    @functools.partial(
        pl.kernel,
        out_type=jax.ShapeDtypeStruct((NC, NPAD, D), jnp.float32),
        mesh=_sc_mesh(),
        scratch_types=[
            pltpu.VMEM((2, NCHUNK, CHUNK), jnp.int32),
            pltpu.VMEM((CHUNK, D), jnp.float32),
            pltpu.VMEM_SHARED((NPAD, D), jnp.float32),
        ],
    )
    def deg_k(ei_hbm, z_hbm, os_hbm, od_hbm, out_hbm, idx_v, buf_v, acc_sh):
        c = lax.axis_index("c")
        s = lax.axis_index("s")
        wid = s * NC + c
        pltpu.sync_copy(ei_hbm.at[0, wid], idx_v.at[0])
        pltpu.sync_copy(ei_hbm.at[1, wid], idx_v.at[1])
        pltpu.sync_copy(z_hbm, buf_v)
        row0 = s * TROWS

        def zbody(k, carry):
            pltpu.sync_copy(buf_v, acc_sh.at[pl.ds(row0 + k * CHUNK, CHUNK)])
            return carry

        lax.fori_loop(0, TROWS // CHUNK, zbody, 0)
        plsc.subcore_barrier()

        pltpu.sync_copy(os_hbm, buf_v)

        def sbody(j, carry):
            pltpu.sync_copy(buf_v, acc_sh.at[idx_v.at[0, j]], add=True)
            return carry

        lax.fori_loop(0, NCHUNK, sbody, 0)
        pltpu.sync_copy(od_hbm, buf_v)

        def dbody(j, carry):
            pltpu.sync_copy(buf_v, acc_sh.at[idx_v.at[1, j]], add=True)
            return carry

        lax.fori_loop(0, NCHUNK, dbody, 0)
        plsc.subcore_barrier()

        def wbody(k, carry):
            pltpu.sync_copy(acc_sh.at[pl.ds(row0 + k * CHUNK, CHUNK)], buf_v)
            pltpu.sync_copy(buf_v, out_hbm.at[c, pl.ds(row0 + k * CHUNK, CHUNK)])
            return carry

        lax.fori_loop(0, TROWS // CHUNK, wbody, 0)

    return deg_k(ei, zrow, onesrc, onedst)


def _spmm_pass(hs, ei, zrow):
    @functools.partial(
        pl.kernel,
        out_type=jax.ShapeDtypeStruct((NC, NPAD, D), jnp.float32),
        mesh=_sc_mesh(),
        scratch_types=[
            pltpu.VMEM((2, NCHUNK, CHUNK), jnp.int32),
            pltpu.VMEM((CHUNK, D), jnp.float32),
            pltpu.VMEM_SHARED((NPAD, D), jnp.float32),
            pltpu.SemaphoreType.DMA,
        ],
    )
    def spmm_k(hs_hbm, ei_hbm, z_hbm, out_hbm, idx_v, rows_v, acc_sh, sem):
        c = lax.axis_index("c")
        s = lax.axis_index("s")
        wid = s * NC + c
        pltpu.sync_copy(ei_hbm.at[0, wid], idx_v.at[0])
        pltpu.sync_copy(ei_hbm.at[1, wid], idx_v.at[1])
        pltpu.sync_copy(z_hbm, rows_v)
        row0 = s * TROWS

        def zbody(k, carry):
            pltpu.sync_copy(rows_v, acc_sh.at[pl.ds(row0 + k * CHUNK, CHUNK)])
            return carry

        lax.fori_loop(0, TROWS // CHUNK, zbody, 0)
        plsc.subcore_barrier()

        def body(j, carry):
            pltpu.async_copy(hs_hbm.at[idx_v.at[0, j]], rows_v, sem).wait()
            pltpu.sync_copy(rows_v, acc_sh.at[idx_v.at[1, j]], add=True)
            return carry

        lax.fori_loop(0, NCHUNK, body, 0)
        plsc.subcore_barrier()

        def wbody(k, carry):
            pltpu.sync_copy(acc_sh.at[pl.ds(row0 + k * CHUNK, CHUNK)], rows_v)
            pltpu.sync_copy(rows_v, out_hbm.at[c, pl.ds(row0 + k * CHUNK, CHUNK)])
            return carry

        lax.fori_loop(0, TROWS // CHUNK, wbody, 0)

    return spmm_k(hs, ei, zrow)


_BLK = 1280
_G = NPAD // _BLK


def _prep_pass(degp, h_pad):
    def body(degp_ref, h_ref, hs_ref, norms_ref):
        deg = degp_ref[0] + degp_ref[1]
        norm = lax.rsqrt(jnp.maximum(deg, 1.0))
        norms_ref[...] = norm
        hs_ref[...] = h_ref[...] * norm[:, 0:1]

    return pl.pallas_call(
        body,
        grid=(_G,),
        in_specs=[
            pl.BlockSpec((NC, _BLK, D), lambda i: (0, i, 0)),
            pl.BlockSpec((_BLK, D), lambda i: (i, 0)),
        ],
        out_specs=[
            pl.BlockSpec((_BLK, D), lambda i: (i, 0)),
            pl.BlockSpec((_BLK, D), lambda i: (i, 0)),
        ],
        out_shape=[
            jax.ShapeDtypeStruct((NPAD, D), jnp.float32),
            jax.ShapeDtypeStruct((NPAD, D), jnp.float32),
        ],
    )(degp, h_pad)


def _mid_pass(p, norms, W1, b1r):
    def body(p_ref, n_ref, w_ref, b_ref, o_ref):
        i = pl.program_id(0)
        agg = p_ref[0] + p_ref[1]
        hd = agg * n_ref[:, 1:2]
        x = jnp.dot(hd, w_ref[...], preferred_element_type=jnp.float32)
        x = jnp.maximum(x + b_ref[...], 0.0)
        rid = lax.broadcasted_iota(jnp.int32, (_BLK, D), 0) + i * _BLK
        x = jnp.where(rid < N, x, 0.0)
        o_ref[...] = x * n_ref[:, 0:1]

    return pl.pallas_call(
        body,
        grid=(_G,),
        in_specs=[
            pl.BlockSpec((NC, _BLK, D), lambda i: (0, i, 0)),
            pl.BlockSpec((_BLK, D), lambda i: (i, 0)),
            pl.BlockSpec((D, D), lambda i: (0, 0)),
            pl.BlockSpec((1, D), lambda i: (0, 0)),
        ],
        out_specs=pl.BlockSpec((_BLK, D), lambda i: (i, 0)),
        out_shape=jax.ShapeDtypeStruct((NPAD, D), jnp.float32),
    )(p, norms, W1, b1r)


def _final_pass(q, norms, W2, b2r, wrT, brr):
    def body(q_ref, n_ref, w_ref, b_ref, wr_ref, br_ref, o_ref, acc_ref):
        i = pl.program_id(0)

        @pl.when(i == 0)
        def _zero():
            acc_ref[...] = jnp.zeros_like(acc_ref)

        agg = q_ref[0] + q_ref[1]
        hd = agg * n_ref[:, 1:2]
        x = jnp.dot(hd, w_ref[...], preferred_element_type=jnp.float32)
        x = jnp.maximum(x + b_ref[...], 0.0)
        rid = lax.broadcasted_iota(jnp.int32, (_BLK, D), 0) + i * _BLK
        x = jnp.where(rid < N, x, 0.0)
        acc_ref[...] += jnp.sum(x, axis=0, keepdims=True)

        @pl.when(i == _G - 1)
        def _readout():
            hg = acc_ref[...] * np.float32(1.0 / N)
            o_ref[...] = jnp.sum(hg * wr_ref[...], axis=1, keepdims=True) + br_ref[...]

    return pl.pallas_call(
        body,
        grid=(_G,),
        in_specs=[
            pl.BlockSpec((NC, _BLK, D), lambda i: (0, i, 0)),
            pl.BlockSpec((_BLK, D), lambda i: (i, 0)),
            pl.BlockSpec((D, D), lambda i: (0, 0)),
            pl.BlockSpec((1, D), lambda i: (0, 0)),
            pl.BlockSpec((1, D), lambda i: (0, 0)),
            pl.BlockSpec((1, 1), lambda i: (0, 0)),
        ],
        out_specs=pl.BlockSpec((1, 1), lambda i: (0, 0)),
        out_shape=jax.ShapeDtypeStruct((1, 1), jnp.float32),
        scratch_shapes=[pltpu.VMEM((1, D), jnp.float32)],
    )(q, norms, W2, b2r, wrT, brr)


def kernel(h, edge_index, W1, b1, W2, b2, Wr, br):
    h_pad = jnp.pad(h, ((0, NPAD - N), (0, 0)))
    pad = jnp.full((2, EPAD - E), DUMMY, jnp.int32)
    ei = jnp.concatenate([edge_index, pad], axis=1).reshape(2, NW, NCHUNK, CHUNK)

    zrow = jnp.zeros((CHUNK, D), jnp.float32)
    col = jnp.arange(D, dtype=jnp.int32)
    onesrc = jnp.broadcast_to((col == 0).astype(jnp.float32), (CHUNK, D))
    onedst = jnp.broadcast_to((col == 1).astype(jnp.float32), (CHUNK, D))

    degp = _deg_pass(ei, zrow, onesrc, onedst)
    hs1, norms = _prep_pass(degp, h_pad)
    p1 = _spmm_pass(hs1, ei, zrow)
    hs2 = _mid_pass(p1, norms, W1, b1.reshape(1, D))
    p2 = _spmm_pass(hs2, ei, zrow)
    return _final_pass(p2, norms, W2, b2.reshape(1, D),
                       Wr.reshape(1, D), br.reshape(1, 1))


# speedup vs baseline: 4.7484x; 1.1792x over previous
import functools

import jax
import jax.numpy as jnp
import numpy as np
from jax import lax
from jax.experimental import pallas as pl
from jax.experimental.pallas import tpu as pltpu
from jax.experimental.pallas import tpu_sc as plsc

N = 10000
E = 320000
D = 128

NC = 2
NS = 16
NW = NC * NS
CHUNK = 128
NCHUNK = 79
EPW = NCHUNK * CHUNK
EPAD = NW * EPW
NPAD = 10240
TROWS = NPAD // NS
DUMMY = NPAD - 1


def _sc_mesh():
    return plsc.VectorSubcoreMesh(core_axis_name="c", subcore_axis_name="s",
                                  num_cores=NC, num_subcores=NS)


def _deg_pass(ei, zrow, onesrc, onedst):
    @functools.partial(
        pl.kernel,
        out_type=jax.ShapeDtypeStruct((NC, NPAD, D), jnp.float32),
        mesh=_sc_mesh(),
        scratch_types=[
            pltpu.VMEM((2, NCHUNK, CHUNK), jnp.int32),
            pltpu.VMEM((CHUNK, D), jnp.float32),
            pltpu.VMEM_SHARED((NPAD, D), jnp.float32),
        ],
    )
    def deg_k(ei_hbm, z_hbm, os_hbm, od_hbm, out_hbm, idx_v, buf_v, acc_sh):
        c = lax.axis_index("c")
        s = lax.axis_index("s")
        wid = s * NC + c
        pltpu.sync_copy(ei_hbm.at[0, wid], idx_v.at[0])
        pltpu.sync_copy(ei_hbm.at[1, wid], idx_v.at[1])
        pltpu.sync_copy(z_hbm, buf_v)
        row0 = s * TROWS

        def zbody(k, carry):
            pltpu.sync_copy(buf_v, acc_sh.at[pl.ds(row0 + k * CHUNK, CHUNK)])
            return carry

        lax.fori_loop(0, TROWS // CHUNK, zbody, 0)
        plsc.subcore_barrier()

        pltpu.sync_copy(os_hbm, buf_v)

        def sbody(j, carry):
            pltpu.sync_copy(buf_v, acc_sh.at[idx_v.at[0, j]], add=True)
            return carry

        lax.fori_loop(0, NCHUNK, sbody, 0)
        pltpu.sync_copy(od_hbm, buf_v)

        def dbody(j, carry):
            pltpu.sync_copy(buf_v, acc_sh.at[idx_v.at[1, j]], add=True)
            return carry

        lax.fori_loop(0, NCHUNK, dbody, 0)
        plsc.subcore_barrier()

        def wbody(k, carry):
            pltpu.sync_copy(acc_sh.at[pl.ds(row0 + k * CHUNK, CHUNK)], buf_v)
            pltpu.sync_copy(buf_v, out_hbm.at[c, pl.ds(row0 + k * CHUNK, CHUNK)])
            return carry

        lax.fori_loop(0, TROWS // CHUNK, wbody, 0)

    return deg_k(ei, zrow, onesrc, onedst)


def _spmm_pass(hs, ei, zrow):
    @functools.partial(
        pl.kernel,
        out_type=jax.ShapeDtypeStruct((NC, NPAD, D), jnp.float32),
        mesh=_sc_mesh(),
        scratch_types=[
            pltpu.VMEM((NCHUNK, CHUNK), jnp.int32),
            pltpu.VMEM((2, CHUNK), jnp.int32),
            pltpu.VMEM((2, CHUNK, D), jnp.float32),
            pltpu.SemaphoreType.DMA((2,)),
            pltpu.SemaphoreType.DMA((2,)),
            pltpu.VMEM_SHARED((NPAD, D), jnp.float32),
        ],
    )
    def spmm_k(hs_hbm, ei_hbm, z_hbm, out_hbm, sidx_v, didx_v, rows_v, gsem, isem, acc_sh):
        c = lax.axis_index("c")
        s = lax.axis_index("s")
        wid = s * NC + c
        pltpu.sync_copy(ei_hbm.at[0, wid], sidx_v)
        pltpu.sync_copy(z_hbm, rows_v.at[0])
        row0 = s * TROWS

        def zbody(k, carry):
            pltpu.sync_copy(rows_v.at[0], acc_sh.at[pl.ds(row0 + k * CHUNK, CHUNK)])
            return carry

        lax.fori_loop(0, TROWS // CHUNK, zbody, 0)
        plsc.subcore_barrier()

        pltpu.async_copy(ei_hbm.at[1, wid, 0], didx_v.at[0], isem.at[0])
        pltpu.async_copy(hs_hbm.at[sidx_v.at[0]], rows_v.at[0], gsem.at[0])

        def body(j, carry):
            b = lax.rem(j, 2)
            nb = lax.rem(j + 1, 2)

            @pl.when(j + 1 < NCHUNK)
            def _prefetch():
                pltpu.async_copy(ei_hbm.at[1, wid, j + 1], didx_v.at[nb],
                                 isem.at[nb])
                pltpu.async_copy(hs_hbm.at[sidx_v.at[j + 1]], rows_v.at[nb],
                                 gsem.at[nb])

            pltpu.make_async_copy(hs_hbm.at[sidx_v.at[j]], rows_v.at[b],
                                  gsem.at[b]).wait()
            pltpu.make_async_copy(ei_hbm.at[1, wid, j], didx_v.at[b],
                                  isem.at[b]).wait()
            pltpu.sync_copy(rows_v.at[b], acc_sh.at[didx_v.at[b]], add=True)
            return carry

        lax.fori_loop(0, NCHUNK, body, 0)
        plsc.subcore_barrier()

        def wbody(k, carry):
            pltpu.sync_copy(acc_sh.at[pl.ds(row0 + k * CHUNK, CHUNK)], rows_v.at[0])
            pltpu.sync_copy(rows_v.at[0], out_hbm.at[c, pl.ds(row0 + k * CHUNK, CHUNK)])
            return carry

        lax.fori_loop(0, TROWS // CHUNK, wbody, 0)

    return spmm_k(hs, ei, zrow)


_BLK = 1280
_G = NPAD // _BLK


def _prep_pass(degp, h_pad):
    def body(degp_ref, h_ref, hs_ref, norms_ref):
        deg = degp_ref[0] + degp_ref[1]
        norm = lax.rsqrt(jnp.maximum(deg, 1.0))
        norms_ref[...] = norm
        hs_ref[...] = h_ref[...] * norm[:, 0:1]

    return pl.pallas_call(
        body,
        grid=(_G,),
        in_specs=[
            pl.BlockSpec((NC, _BLK, D), lambda i: (0, i, 0)),
            pl.BlockSpec((_BLK, D), lambda i: (i, 0)),
        ],
        out_specs=[
            pl.BlockSpec((_BLK, D), lambda i: (i, 0)),
            pl.BlockSpec((_BLK, D), lambda i: (i, 0)),
        ],
        out_shape=[
            jax.ShapeDtypeStruct((NPAD, D), jnp.float32),
            jax.ShapeDtypeStruct((NPAD, D), jnp.float32),
        ],
    )(degp, h_pad)


def _mid_pass(p, norms, W1, b1r):
    def body(p_ref, n_ref, w_ref, b_ref, o_ref):
        i = pl.program_id(0)
        agg = p_ref[0] + p_ref[1]
        hd = agg * n_ref[:, 1:2]
        x = jnp.dot(hd, w_ref[...], preferred_element_type=jnp.float32)
        x = jnp.maximum(x + b_ref[...], 0.0)
        rid = lax.broadcasted_iota(jnp.int32, (_BLK, D), 0) + i * _BLK
        x = jnp.where(rid < N, x, 0.0)
        o_ref[...] = x * n_ref[:, 0:1]

    return pl.pallas_call(
        body,
        grid=(_G,),
        in_specs=[
            pl.BlockSpec((NC, _BLK, D), lambda i: (0, i, 0)),
            pl.BlockSpec((_BLK, D), lambda i: (i, 0)),
            pl.BlockSpec((D, D), lambda i: (0, 0)),
            pl.BlockSpec((1, D), lambda i: (0, 0)),
        ],
        out_specs=pl.BlockSpec((_BLK, D), lambda i: (i, 0)),
        out_shape=jax.ShapeDtypeStruct((NPAD, D), jnp.float32),
    )(p, norms, W1, b1r)


def _final_pass(q, norms, W2, b2r, wrT, brr):
    def body(q_ref, n_ref, w_ref, b_ref, wr_ref, br_ref, o_ref, acc_ref):
        i = pl.program_id(0)

        @pl.when(i == 0)
        def _zero():
            acc_ref[...] = jnp.zeros_like(acc_ref)

        agg = q_ref[0] + q_ref[1]
        hd = agg * n_ref[:, 1:2]
        x = jnp.dot(hd, w_ref[...], preferred_element_type=jnp.float32)
        x = jnp.maximum(x + b_ref[...], 0.0)
        rid = lax.broadcasted_iota(jnp.int32, (_BLK, D), 0) + i * _BLK
        x = jnp.where(rid < N, x, 0.0)
        acc_ref[...] += jnp.sum(x, axis=0, keepdims=True)

        @pl.when(i == _G - 1)
        def _readout():
            hg = acc_ref[...] * np.float32(1.0 / N)
            o_ref[...] = jnp.sum(hg * wr_ref[...], axis=1, keepdims=True) + br_ref[...]

    return pl.pallas_call(
        body,
        grid=(_G,),
        in_specs=[
            pl.BlockSpec((NC, _BLK, D), lambda i: (0, i, 0)),
            pl.BlockSpec((_BLK, D), lambda i: (i, 0)),
            pl.BlockSpec((D, D), lambda i: (0, 0)),
            pl.BlockSpec((1, D), lambda i: (0, 0)),
            pl.BlockSpec((1, D), lambda i: (0, 0)),
            pl.BlockSpec((1, 1), lambda i: (0, 0)),
        ],
        out_specs=pl.BlockSpec((1, 1), lambda i: (0, 0)),
        out_shape=jax.ShapeDtypeStruct((1, 1), jnp.float32),
        scratch_shapes=[pltpu.VMEM((1, D), jnp.float32)],
    )(q, norms, W2, b2r, wrT, brr)


def kernel(h, edge_index, W1, b1, W2, b2, Wr, br):
    h_pad = jnp.pad(h, ((0, NPAD - N), (0, 0)))
    pad = jnp.full((2, EPAD - E), DUMMY, jnp.int32)
    ei = jnp.concatenate([edge_index, pad], axis=1).reshape(2, NW, NCHUNK, CHUNK)

    zrow = jnp.zeros((CHUNK, D), jnp.float32)
    col = jnp.arange(D, dtype=jnp.int32)
    onesrc = jnp.broadcast_to((col == 0).astype(jnp.float32), (CHUNK, D))
    onedst = jnp.broadcast_to((col == 1).astype(jnp.float32), (CHUNK, D))

    degp = _deg_pass(ei, zrow, onesrc, onedst)
    hs1, norms = _prep_pass(degp, h_pad)
    p1 = _spmm_pass(hs1, ei, zrow)
    hs2 = _mid_pass(p1, norms, W1, b1.reshape(1, D))
    p2 = _spmm_pass(hs2, ei, zrow)
    return _final_pass(p2, norms, W2, b2.reshape(1, D),
                       Wr.reshape(1, D), br.reshape(1, 1))
